# Initial kernel scaffold; baseline (speedup 1.0000x reference)
import jax
import jax.numpy as jnp
from jax.experimental import pallas as pl


def kernel(params, nfeats, efeats, edge_index, node2graph):
    raise NotImplementedError("write your pallas kernel here")


# speedup vs baseline: 2.8790x; 2.8790x over previous
import functools

import jax
import jax.numpy as jnp
from jax import lax
from jax.experimental import pallas as pl
from jax.experimental.pallas import tpu as pltpu
from jax.experimental.pallas import tpu_sc as plsc

_N = 10000
_NP = 10240
_E = 160000
_H = 160
_NB = 1024
_K = 80
_NC = 2
_NS = 16
_EPS = _E // _NS
_RPS = _NP // _NS
_GRID = _NP // _NB
_F32 = jnp.float32

def _mesh():
    return plsc.VectorSubcoreMesh(
        core_axis_name="c", subcore_axis_name="s",
        num_cores=_NC, num_subcores=_NS)


def _sc_segsum_body(hs, srcs, dst, out, acc, sv, dv, rows, sem):
    c = lax.axis_index("c")
    s = lax.axis_index("s")
    rowbase = c * _NP + s * _RPS
    accbase = s * _RPS
    pltpu.sync_copy(hs.at[pl.ds(rowbase, _RPS)], acc.at[pl.ds(accbase, _RPS)])
    plsc.subcore_barrier()
    ebase = s * _EPS

    def _step(i, carry):
        off = ebase + i * _K
        pltpu.sync_copy(srcs.at[pl.ds(c * _E + off, _K)], sv)
        pltpu.sync_copy(dst.at[pl.ds(off, _K)], dv)
        pltpu.async_copy(hs.at[sv], rows, sem).wait()
        pltpu.sync_copy(rows, acc.at[dv], add=True)
        return carry

    lax.fori_loop(0, _EPS // _K, _step, 0)
    plsc.subcore_barrier()
    pltpu.sync_copy(acc.at[pl.ds(accbase, _RPS)], out.at[pl.ds(rowbase, _RPS)])


def _sc_segsum(hs, srcs, dst):
    return pl.kernel(
        _sc_segsum_body,
        out_type=jax.ShapeDtypeStruct((2 * _NP, _H), _F32),
        mesh=_mesh(),
        compiler_params=pltpu.CompilerParams(use_tc_tiling_on_sc=False),
        scratch_types=[
            pltpu.VMEM_SHARED((_NP, _H), _F32),
            pltpu.VMEM((_K,), jnp.int32),
            pltpu.VMEM((_K,), jnp.int32),
            pltpu.VMEM((_K, _H), _F32),
            pltpu.SemaphoreType.DMA,
        ],
    )(hs, srcs, dst)


def _sc_counts_body(efs, dst, eye, out, acc, zb, ev, dv, rows, sem):
    c = lax.axis_index("c")
    s = lax.axis_index("s")
    accbase = s * _RPS

    def _zero(i, carry):
        zb[i, :] = jnp.zeros((16,), _F32)
        return carry

    lax.fori_loop(0, _RPS, _zero, 0)
    pltpu.sync_copy(zb, acc.at[pl.ds(accbase, _RPS)])
    plsc.subcore_barrier()
    ebase = s * _EPS

    def _step(i, carry):
        off = ebase + i * _K
        pltpu.sync_copy(efs.at[pl.ds(c * _E + off, _K)], ev)
        pltpu.sync_copy(dst.at[pl.ds(off, _K)], dv)
        pltpu.async_copy(eye.at[ev], rows, sem).wait()
        pltpu.sync_copy(rows, acc.at[dv], add=True)
        return carry

    lax.fori_loop(0, _EPS // _K, _step, 0)
    plsc.subcore_barrier()
    pltpu.sync_copy(acc.at[pl.ds(accbase, _RPS)],
                    out.at[pl.ds(c * _NP + accbase, _RPS)])


def _sc_counts(efs, dst, eye):
    return pl.kernel(
        _sc_counts_body,
        out_type=jax.ShapeDtypeStruct((2 * _NP, 16), _F32),
        mesh=_mesh(),
        compiler_params=pltpu.CompilerParams(use_tc_tiling_on_sc=False),
        scratch_types=[
            pltpu.VMEM_SHARED((_NP, 16), _F32),
            pltpu.VMEM((_RPS, 16), _F32),
            pltpu.VMEM((_K,), jnp.int32),
            pltpu.VMEM((_K,), jnp.int32),
            pltpu.VMEM((_K, 16), _F32),
            pltpu.SemaphoreType.DMA,
        ],
    )(efs, dst, eye)


def _embed_body(nf, tbl, out):
    n0 = nf[:, 0:1]
    n1 = nf[:, 1:2] + 120
    iot = lax.broadcasted_iota(jnp.int32, (_NB, 128), 1)
    oh = (iot == n0).astype(_F32) + (iot == n1).astype(_F32)
    for k in range(2):
        out[k] = jnp.dot(oh, tbl[k], preferred_element_type=_F32, precision=lax.Precision.HIGHEST)


def _embed(nf, tbl):
    return pl.pallas_call(
        _embed_body,
        grid=(_GRID,),
        in_specs=[
            pl.BlockSpec((_NB, 2), lambda i: (i, 0)),
            pl.BlockSpec((2, 128, _H), lambda i: (0, 0, 0)),
        ],
        out_specs=pl.BlockSpec((2, _NB, _H), lambda i: (0, i, 0)),
        out_shape=jax.ShapeDtypeStruct((2, _NP, _H), _F32),
    )(nf, tbl)


def _mlp_body(s3, c2, we, w1, b1, w2, b2, z, sums):
    i = pl.program_id(0)
    cb = c2[0] + c2[1]
    acc = jnp.zeros((_NB, 608), _F32)
    for k in range(2):
        aggk = s3[k] + jnp.dot(cb, we[k], preferred_element_type=_F32, precision=lax.Precision.HIGHEST)
        acc = acc + jnp.dot(aggk, w1[k], preferred_element_type=_F32, precision=lax.Precision.HIGHEST)
    y = jnp.maximum(acc + b1[...], 0.0)
    rmask = ((lax.broadcasted_iota(jnp.int32, (_NB, 1), 0) + i * _NB)
             < _N).astype(_F32)
    parts = []
    for k in range(2):
        zk = (jnp.dot(y, w2[k], preferred_element_type=_F32, precision=lax.Precision.HIGHEST) + b2[k]) * rmask
        z[k] = zk
        s1 = jnp.sum(zk, axis=0, keepdims=True)
        s2 = jnp.sum(zk * zk, axis=0, keepdims=True)
        parts.append(jnp.concatenate(
            [s1, s2, jnp.zeros((6, _H), _F32)], axis=0))
    upd = jnp.stack(parts)

    @pl.when(i == 0)
    def _():
        sums[...] = upd

    @pl.when(i > 0)
    def _():
        sums[...] = sums[...] + upd


def _mlp(s3, c2, we, w1, b1, w2, b2):
    return pl.pallas_call(
        _mlp_body,
        grid=(_GRID,),
        in_specs=[
            pl.BlockSpec((2, _NB, _H), lambda i: (0, i, 0)),
            pl.BlockSpec((2, _NB, 16), lambda i: (0, i, 0)),
            pl.BlockSpec((2, 16, _H), lambda i: (0, 0, 0)),
            pl.BlockSpec((2, _H, 608), lambda i: (0, 0, 0)),
            pl.BlockSpec((1, 608), lambda i: (0, 0)),
            pl.BlockSpec((2, 608, _H), lambda i: (0, 0, 0)),
            pl.BlockSpec((2, 1, _H), lambda i: (0, 0, 0)),
        ],
        out_specs=[
            pl.BlockSpec((2, _NB, _H), lambda i: (0, i, 0)),
            pl.BlockSpec((2, 8, _H), lambda i: (0, 0, 0)),
        ],
        out_shape=[
            jax.ShapeDtypeStruct((2, _NP, _H), _F32),
            jax.ShapeDtypeStruct((2, 8, _H), _F32),
        ],
    )(s3, c2, we, w1, b1, w2, b2)


def _bn_body(z, sums, gp, bp, out, vacc, *, relu):
    p = pl.program_id(0)
    i = pl.program_id(1)
    rmask = ((lax.broadcasted_iota(jnp.int32, (_NB, 1), 0) + i * _NB)
             < _N).astype(_F32)

    @pl.when((p == 0) & (i == 0))
    def _():
        vacc[...] = jnp.zeros_like(vacc)

    @pl.when(p == 0)
    def _():
        for k in range(2):
            mean = sums[k, 0:1, :] / _N
            d = (z[k] - mean) * rmask
            vacc[k] = vacc[k] + jnp.sum(d * d, axis=0, keepdims=True)

    @pl.when(p == 1)
    def _():
        for k in range(2):
            mean = sums[k, 0:1, :] / _N
            var = vacc[k] / _N
            inv = lax.rsqrt(var + 1e-5)
            zz = (z[k] - mean) * (inv * gp[k]) + bp[k]
            if relu:
                zz = jnp.maximum(zz, 0.0)
            out[k] = zz


def _bn(z3, sums, gp, bp, relu):
    return pl.pallas_call(
        functools.partial(_bn_body, relu=relu),
        grid=(2, _GRID),
        in_specs=[
            pl.BlockSpec((2, _NB, _H), lambda p, i: (0, i, 0)),
            pl.BlockSpec((2, 8, _H), lambda p, i: (0, 0, 0)),
            pl.BlockSpec((2, 1, _H), lambda p, i: (0, 0, 0)),
            pl.BlockSpec((2, 1, _H), lambda p, i: (0, 0, 0)),
        ],
        out_specs=pl.BlockSpec((2, _NB, _H), lambda p, i: (0, i, 0)),
        out_shape=jax.ShapeDtypeStruct((2, _NP, _H), _F32),
        scratch_shapes=[pltpu.VMEM((2, 1, _H), _F32)],
    )(z3, sums, gp, bp)


def _readout_body(h3, n2g, wa, ba, wb, bb, wc, bc, out, gacc, cacc):
    i = pl.program_id(0)

    @pl.when(i == 0)
    def _():
        gacc[...] = jnp.zeros_like(gacc)
        cacc[...] = jnp.zeros_like(cacc)

    iot = lax.broadcasted_iota(jnp.int32, (_NB, 128), 1)
    oh = (iot == n2g[...]).astype(_F32)
    dnum = (((0,), (0,)), ((), ()))
    for k in range(2):
        gacc[k] = gacc[k] + lax.dot_general(
            oh, h3[k], dnum, preferred_element_type=_F32, precision=lax.Precision.HIGHEST)
    cacc[...] = cacc[...] + lax.dot_general(
        oh, jnp.ones((_NB, 8), _F32), dnum, preferred_element_type=_F32, precision=lax.Precision.HIGHEST)

    @pl.when(i == _GRID - 1)
    def _():
        cnt = jnp.maximum(cacc[:, 0:1], 1.0)
        a = ba[...]
        a = jnp.zeros((128, 128), _F32)
        for k in range(2):
            a = a + jnp.dot(gacc[k] / cnt, wa[k], preferred_element_type=_F32, precision=lax.Precision.HIGHEST)
        a = jnp.maximum(a + ba[...], 0.0)
        a = jnp.maximum(jnp.dot(a, wb[...], preferred_element_type=_F32, precision=lax.Precision.HIGHEST)
                        + bb[...], 0.0)
        o = jnp.dot(a, wc[...], preferred_element_type=_F32, precision=lax.Precision.HIGHEST) + bc[...]
        out[...] = o[:64, 0:1]


def _readout(h3, n2g, wa, ba, wb, bb, wc, bc):
    return pl.pallas_call(
        _readout_body,
        grid=(_GRID,),
        in_specs=[
            pl.BlockSpec((2, _NB, _H), lambda i: (0, i, 0)),
            pl.BlockSpec((_NB, 1), lambda i: (i, 0)),
            pl.BlockSpec((2, _H, 128), lambda i: (0, 0, 0)),
            pl.BlockSpec((1, 128), lambda i: (0, 0)),
            pl.BlockSpec((128, 32), lambda i: (0, 0)),
            pl.BlockSpec((1, 32), lambda i: (0, 0)),
            pl.BlockSpec((32, 128), lambda i: (0, 0)),
            pl.BlockSpec((1, 128), lambda i: (0, 0)),
        ],
        out_specs=pl.BlockSpec((64, 1), lambda i: (0, 0)),
        out_shape=jax.ShapeDtypeStruct((64, 1), _F32),
        scratch_shapes=[
            pltpu.VMEM((2, 128, _H), _F32),
            pltpu.VMEM((128, 8), _F32),
        ],
    )(h3, n2g, wa, ba, wb, bb, wc, bc)


def _split_cols(m, width=_H):
    r = m.shape[0]
    o = jnp.zeros((2, r, width), _F32)
    o = o.at[0, :, :150].set(m[:, :150])
    o = o.at[1, :, :150].set(m[:, 150:300])
    return o


def _split_vec(v, width=_H):
    return _split_cols(v.reshape(1, 300), width)


def kernel(params, nfeats, efeats, edge_index, node2graph):
    nfeats = nfeats.astype(jnp.int32)
    efeats = efeats.astype(jnp.int32)
    edge_index = edge_index.astype(jnp.int32)
    node2graph = node2graph.astype(jnp.int32)

    ntab = jnp.concatenate([
        params['node_emb0'], params['node_emb1'],
        jnp.zeros((5, 300), _F32)], axis=0)
    tbl = _split_cols(ntab)

    src = edge_index[0]
    dst = edge_index[1]
    srcs = jnp.concatenate([src, src + _NP])
    efs = jnp.concatenate([efeats[:, 0], efeats[:, 1] + 6])
    eye16 = jnp.eye(16, dtype=_F32)
    nfp = jnp.zeros((_NP, 2), jnp.int32).at[:_N].set(nfeats)
    n2gp = jnp.full((_NP,), 127, jnp.int32).at[:_N].set(node2graph)

    h3 = _embed(nfp, tbl)
    c2 = _sc_counts(efs, dst, eye16).reshape(2, _NP, 16)

    for l in range(5):
        p = params['layers'][l]
        etab = jnp.concatenate([
            p['edge_emb0'], p['edge_emb1'],
            jnp.zeros((7, 300), _F32)], axis=0)
        we = _split_cols(etab)
        w1 = jnp.zeros((2, _H, 608), _F32)
        w1 = w1.at[0, :150, :600].set(p['W1'][:150])
        w1 = w1.at[1, :150, :600].set(p['W1'][150:300])
        b1 = jnp.zeros((1, 608), _F32).at[0, :600].set(p['b1'])
        w2 = jnp.zeros((2, 608, _H), _F32)
        w2 = w2.at[0, :600, :150].set(p['W2'][:, :150])
        w2 = w2.at[1, :600, :150].set(p['W2'][:, 150:300])
        b2 = _split_vec(p['b2'])
        gp = _split_vec(p['gamma'])
        bp = _split_vec(p['beta'])

        s3 = _sc_segsum(h3.reshape(2 * _NP, _H), srcs, dst).reshape(2, _NP, _H)
        z3, sums = _mlp(s3, c2, we, w1, b1, w2, b2)
        h3 = _bn(z3, sums, gp, bp, relu=(l < 4))

    wa = jnp.zeros((2, _H, 128), _F32)
    wa = wa.at[0, :150].set(params['Wa'][:150])
    wa = wa.at[1, :150].set(params['Wa'][150:300])
    ba = params['ba'].reshape(1, 128)
    wb = params['Wb']
    bb = params['bb'].reshape(1, 32)
    wc = jnp.zeros((32, 128), _F32).at[:, 0].set(params['Wc'][:, 0])
    bc = jnp.zeros((1, 128), _F32).at[0, 0].set(params['bc'][0])

    return _readout(h3, n2gp.reshape(_NP, 1), wa, ba, wb, bb, wc, bc)
